# baseline (speedup 1.0000x reference)
import functools

import jax
import jax.numpy as jnp
from jax import lax
from jax.experimental import pallas as pl
from jax.experimental.pallas import tpu as pltpu
from jax.experimental.pallas import tpu_sc as plsc

N = 10000
E = 320000
D = 128
DFF = 512
NC, NS = 2, 16
NW = NC * NS
EPW = E // NW
CHUNK = 100
NCHUNK = EPW // CHUNK
CPP = 20
NPH = EPW // (CPP * CHUNK)
NPAD = 10240
DEG_PT = NPAD // NS
NPADR = 10240
ROWS_PT = NPADR // NS
CHUNKR = 80
RB = 2000
NB = N // RB
EPS = 1e-5

@functools.lru_cache(maxsize=None)
def _sc_mesh():
    return plsc.VectorSubcoreMesh(
        core_axis_name="c", subcore_axis_name="s", num_cores=NC, num_subcores=NS)


def _deg_body(dst_hbm, ones_hbm, zeros_hbm, deg_hbm, idx_v, ones_v, zb_v, acc,
              dsem):
    c = lax.axis_index("c")
    s = lax.axis_index("s")
    wid = c * NS + s

    pltpu.sync_copy(zeros_hbm, zb_v)
    pltpu.sync_copy(zb_v, acc.at[pl.ds(s * DEG_PT, DEG_PT)])
    pltpu.sync_copy(dst_hbm.at[wid], idx_v)
    pltpu.sync_copy(ones_hbm.at[pl.ds(0, CHUNK)], ones_v)
    plsc.subcore_barrier()

    def body(j, carry):
        pltpu.async_copy(ones_v, acc.at[idx_v.at[j]], dsem, add=True)
        return carry

    lax.fori_loop(0, NCHUNK, body, 0)

    def drain(j, carry):
        pltpu.make_async_copy(ones_v, acc.at[idx_v.at[j]], dsem).wait()
        return carry

    lax.fori_loop(0, NCHUNK, drain, 0)
    plsc.subcore_barrier()
    pltpu.sync_copy(acc.at[pl.ds(s * DEG_PT, DEG_PT)], zb_v)
    pltpu.sync_copy(zb_v, deg_hbm.at[c, pl.ds(s * DEG_PT, DEG_PT)])


@functools.lru_cache(maxsize=None)
def _deg_kernel_fn():
    return pl.kernel(
        _deg_body,
        out_type=jax.ShapeDtypeStruct((NC, NPAD), jnp.float32),
        mesh=_sc_mesh(),
        scratch_types=[
            pltpu.VMEM((NCHUNK, CHUNK), jnp.int32),
            pltpu.VMEM((CHUNK,), jnp.float32),
            pltpu.VMEM((DEG_PT,), jnp.float32),
            pltpu.VMEM_SHARED((NPAD,), jnp.float32),
            pltpu.SemaphoreType.DMA,
        ],
    )


def _deg_kernel(dst, ones_pad, zeros_deg):
    return _deg_kernel_fn()(dst, ones_pad, zeros_deg)


def _seg_body(src_hbm, dst_hbm, g_hbm, zrow_hbm, out_hbm,
              src_v, dst_v, buf0, buf1, acc, sem0, sem1, ssem0, ssem1):
    c = lax.axis_index("c")
    s = lax.axis_index("s")
    wid = c * NS + s

    wb = buf0.at[pl.ds(0, CHUNKR)]
    pltpu.sync_copy(zrow_hbm, wb)
    for z in range(ROWS_PT // CHUNKR):
        pltpu.sync_copy(wb, acc.at[pl.ds(s * ROWS_PT + z * CHUNKR, CHUNKR)])
    plsc.subcore_barrier()

    for ph in range(NPH):
        blk = wid * NPH + ph
        pltpu.sync_copy(src_hbm.at[blk], src_v)
        pltpu.sync_copy(dst_hbm.at[blk], dst_v)
        pltpu.async_copy(g_hbm.at[src_v.at[0]], buf0, sem0)
        pltpu.async_copy(g_hbm.at[src_v.at[1]], buf1, sem1)

        def body(k, carry):
            j0 = 2 * k
            j1 = 2 * k + 1
            pltpu.make_async_copy(g_hbm.at[src_v.at[j0]], buf0, sem0).wait()
            pltpu.async_copy(buf0, acc.at[dst_v.at[j0]], ssem0, add=True)
            pltpu.make_async_copy(g_hbm.at[src_v.at[j1]], buf1, sem1).wait()
            pltpu.async_copy(buf1, acc.at[dst_v.at[j1]], ssem1, add=True)

            pltpu.make_async_copy(buf0, acc.at[dst_v.at[j0]], ssem0).wait()

            @pl.when(j0 + 2 < CPP)
            def _():
                pltpu.async_copy(g_hbm.at[src_v.at[j0 + 2]], buf0, sem0)

            pltpu.make_async_copy(buf1, acc.at[dst_v.at[j1]], ssem1).wait()

            @pl.when(j1 + 2 < CPP)
            def _():
                pltpu.async_copy(g_hbm.at[src_v.at[j1 + 2]], buf1, sem1)

            return carry

        lax.fori_loop(0, CPP // 2, body, 0)

    plsc.subcore_barrier()
    for z in range(ROWS_PT // CHUNKR):
        r0 = s * ROWS_PT + z * CHUNKR
        pltpu.sync_copy(acc.at[pl.ds(r0, CHUNKR)], wb)
        pltpu.sync_copy(wb, out_hbm.at[c, pl.ds(r0, CHUNKR)])


@functools.lru_cache(maxsize=None)
def _seg_kernel_fn():
    return pl.kernel(
        _seg_body,
        out_type=jax.ShapeDtypeStruct((NC, NPADR, D), jnp.float32),
        mesh=_sc_mesh(),
        scratch_types=[
            pltpu.VMEM((CPP, CHUNK), jnp.int32),
            pltpu.VMEM((CPP, CHUNK), jnp.int32),
            pltpu.VMEM((CHUNK, D), jnp.float32),
            pltpu.VMEM((CHUNK, D), jnp.float32),
            pltpu.VMEM_SHARED((NPADR, D), jnp.float32),
            pltpu.SemaphoreType.DMA,
            pltpu.SemaphoreType.DMA,
            pltpu.SemaphoreType.DMA,
            pltpu.SemaphoreType.DMA,
        ],
    )


def _seg_kernel(src, dst, g, zero_rows):
    return _seg_kernel_fn()(src, dst, g, zero_rows)


def _pre_body(x_ref, w_ref, deg_ref, g_ref, dinv_ref):
    dinv = lax.rsqrt(deg_ref[...])
    h = jnp.dot(x_ref[...], w_ref[...], preferred_element_type=jnp.float32)
    g_ref[...] = h * dinv
    dinv_ref[...] = dinv


def _tc_pre(x, w, deg):
    return pl.pallas_call(
        _pre_body,
        grid=(NB,),
        in_specs=[
            pl.BlockSpec((RB, D), lambda b: (b, 0)),
            pl.BlockSpec((D, D), lambda b: (0, 0)),
            pl.BlockSpec((RB, 1), lambda b: (b, 0)),
        ],
        out_specs=[
            pl.BlockSpec((RB, D), lambda b: (b, 0)),
            pl.BlockSpec((RB, 1), lambda b: (b, 0)),
        ],
        out_shape=[
            jax.ShapeDtypeStruct((N, D), jnp.float32),
            jax.ShapeDtypeStruct((N, 1), jnp.float32),
        ],
    )(x, w, deg)


def _post_body(x_ref, e_ref, g_ref, dinv_ref, bgcn_ref, bnw_ref, bnb_ref,
               w1_ref, b1_ref, w2_ref, b2_ref, out_ref, y_ref, st_ref, cf_ref):
    p = pl.program_id(0)
    b = pl.program_id(1)
    rows = pl.ds(b * RB, RB)

    @pl.when(p == 0)
    def _():
        @pl.when(b == 0)
        def _():
            st_ref[...] = jnp.zeros_like(st_ref)

        e = e_ref[0] + e_ref[1]
        y = x_ref[...] + dinv_ref[...] * (e + g_ref[...]) + bgcn_ref[...]
        y_ref[rows, :] = y
        st_ref[0:1, :] += jnp.sum(y, axis=0, keepdims=True)
        st_ref[1:2, :] += jnp.sum(y * y, axis=0, keepdims=True)

    @pl.when(p == 1)
    def _():
        @pl.when(b == 0)
        def _():
            mean = st_ref[0:1, :] * (1.0 / N)
            var = st_ref[1:2, :] * (1.0 / N) - mean * mean
            sc = bnw_ref[...] * lax.rsqrt(var + EPS)
            cf_ref[0:1, :] = sc
            cf_ref[1:2, :] = bnb_ref[...] - mean * sc

        h1 = y_ref[rows, :] * cf_ref[0:1, :] + cf_ref[1:2, :]
        t = lax.dot_general(h1, w1_ref[...], (((1,), (1,)), ((), ())),
                            preferred_element_type=jnp.float32)
        t = jnp.maximum(t + b1_ref[...], 0.0)
        ff = lax.dot_general(t, w2_ref[...], (((1,), (1,)), ((), ())),
                             preferred_element_type=jnp.float32)
        z = h1 + ff + b2_ref[...]
        y_ref[rows, :] = z
        st_ref[2:3, :] += jnp.sum(z, axis=0, keepdims=True)
        st_ref[3:4, :] += jnp.sum(z * z, axis=0, keepdims=True)

    @pl.when(p == 2)
    def _():
        @pl.when(b == 0)
        def _():
            mean = st_ref[2:3, :] * (1.0 / N)
            var = st_ref[3:4, :] * (1.0 / N) - mean * mean
            sc = bnw_ref[...] * lax.rsqrt(var + EPS)
            cf_ref[2:3, :] = sc
            cf_ref[3:4, :] = bnb_ref[...] - mean * sc

        out_ref[...] = y_ref[rows, :] * cf_ref[2:3, :] + cf_ref[3:4, :]


def _tc_post(x, seg, g, dinv, bgcn, bnw, bnb, w1, b1, w2, b2):
    full = lambda shp: pl.BlockSpec(shp, lambda p, b: tuple(0 for _ in shp))
    return pl.pallas_call(
        _post_body,
        grid=(3, NB),
        in_specs=[
            pl.BlockSpec((RB, D), lambda p, b: (b, 0)),
            pl.BlockSpec((NC, RB, D), lambda p, b: (0, b, 0)),
            pl.BlockSpec((RB, D), lambda p, b: (b, 0)),
            pl.BlockSpec((RB, 1), lambda p, b: (b, 0)),
            full((1, D)),
            full((1, D)),
            full((1, D)),
            full((DFF, D)),
            full((1, DFF)),
            full((D, DFF)),
            full((1, D)),
        ],
        out_specs=pl.BlockSpec((RB, D), lambda p, b: (b, 0)),
        out_shape=jax.ShapeDtypeStruct((N, D), jnp.float32),
        scratch_shapes=[
            pltpu.VMEM((N, D), jnp.float32),
            pltpu.VMEM((8, D), jnp.float32),
            pltpu.VMEM((8, D), jnp.float32),
        ],
    )(x, seg, g, dinv, bgcn, bnw, bnb, w1, b1, w2, b2)


def kernel(x, edge_index, W_gcn, b_gcn, bn_w, bn_b, W1, b1, W2, b2):
    ei = edge_index.astype(jnp.int32)
    src = ei[0].reshape(NW, NCHUNK, CHUNK)
    dst = ei[1].reshape(NW, NCHUNK, CHUNK)
    src2 = ei[0].reshape(NW * NPH, CPP, CHUNK)
    dst2 = ei[1].reshape(NW * NPH, CPP, CHUNK)

    ones_pad = jnp.ones((128,), jnp.float32)
    zeros_deg = jnp.zeros((DEG_PT,), jnp.float32)
    zero_rows = jnp.zeros((CHUNKR, D), jnp.float32)

    deg_p = _deg_kernel(dst, ones_pad, zeros_deg)
    deg = (deg_p[0, :N] + deg_p[1, :N] + 1.0).reshape(N, 1)

    g, dinv = _tc_pre(x, W_gcn, deg)
    seg = _seg_kernel(src2, dst2, g, zero_rows)

    return _tc_post(x, seg, g, dinv,
                    b_gcn.reshape(1, D), bn_w.reshape(1, D), bn_b.reshape(1, D),
                    W1, b1.reshape(1, DFF), W2, b2.reshape(1, D))


# speedup vs baseline: 1.1571x; 1.1571x over previous
import functools

import jax
import jax.numpy as jnp
from jax import lax
from jax.experimental import pallas as pl
from jax.experimental.pallas import tpu as pltpu
from jax.experimental.pallas import tpu_sc as plsc

N = 10000
E = 320000
D = 128
DFF = 512
NC, NS = 2, 16
NW = NC * NS
EPW = E // NW
CHUNK = 100
NCHUNK = EPW // CHUNK
CPP = 20
NPH = EPW // (CPP * CHUNK)
NPAD = 10240
DEG_PT = NPAD // NS
NPADR = 10240
ROWS_PT = NPADR // NS
CHUNKR = 80
RB = 2000
NB = N // RB
EPS = 1e-5

@functools.lru_cache(maxsize=None)
def _sc_mesh():
    return plsc.VectorSubcoreMesh(
        core_axis_name="c", subcore_axis_name="s", num_cores=NC, num_subcores=NS)


def _deg_body(dst_hbm, ones_hbm, zeros_hbm, deg_hbm, idx_v, ones_v, zb_v, acc,
              dsem):
    c = lax.axis_index("c")
    s = lax.axis_index("s")
    wid = c * NS + s

    pltpu.sync_copy(zeros_hbm, zb_v)
    pltpu.sync_copy(zb_v, acc.at[pl.ds(s * DEG_PT, DEG_PT)])
    pltpu.sync_copy(dst_hbm.at[wid], idx_v)
    pltpu.sync_copy(ones_hbm.at[pl.ds(0, CHUNK)], ones_v)
    plsc.subcore_barrier()

    def body(j, carry):
        pltpu.async_copy(ones_v, acc.at[idx_v.at[j]], dsem, add=True)
        return carry

    lax.fori_loop(0, NCHUNK, body, 0)

    def drain(j, carry):
        pltpu.make_async_copy(ones_v, acc.at[idx_v.at[j]], dsem).wait()
        return carry

    lax.fori_loop(0, NCHUNK, drain, 0)
    plsc.subcore_barrier()
    pltpu.sync_copy(acc.at[pl.ds(s * DEG_PT, DEG_PT)], zb_v)
    pltpu.sync_copy(zb_v, deg_hbm.at[c, pl.ds(s * DEG_PT, DEG_PT)])


@functools.lru_cache(maxsize=None)
def _deg_kernel_fn():
    return pl.kernel(
        _deg_body,
        out_type=jax.ShapeDtypeStruct((NC, NPAD), jnp.float32),
        mesh=_sc_mesh(),
        scratch_types=[
            pltpu.VMEM((NCHUNK, CHUNK), jnp.int32),
            pltpu.VMEM((CHUNK,), jnp.float32),
            pltpu.VMEM((DEG_PT,), jnp.float32),
            pltpu.VMEM_SHARED((NPAD,), jnp.float32),
            pltpu.SemaphoreType.DMA,
        ],
    )


def _deg_kernel(dst, ones_pad, zeros_deg):
    return _deg_kernel_fn()(dst, ones_pad, zeros_deg)


def _seg_body(src_hbm, dst_hbm, g_hbm, zrow_hbm, out_hbm,
              src_v, dst_v, buf0, buf1, acc, sem0, sem1):
    c = lax.axis_index("c")
    s = lax.axis_index("s")
    wid = c * NS + s

    wb = buf0.at[pl.ds(0, CHUNKR)]
    pltpu.sync_copy(zrow_hbm, wb)
    for z in range(ROWS_PT // CHUNKR):
        pltpu.sync_copy(wb, acc.at[pl.ds(s * ROWS_PT + z * CHUNKR, CHUNKR)])
    plsc.subcore_barrier()

    for ph in range(NPH):
        blk = wid * NPH + ph
        pltpu.sync_copy(src_hbm.at[blk], src_v)
        pltpu.sync_copy(dst_hbm.at[blk], dst_v)
        pltpu.async_copy(g_hbm.at[src_v.at[0]], buf0, sem0)
        pltpu.async_copy(g_hbm.at[src_v.at[1]], buf1, sem1)

        def body(k, carry):
            j0 = 2 * k
            j1 = 2 * k + 1
            pltpu.make_async_copy(g_hbm.at[src_v.at[j0]], buf0, sem0).wait()
            pltpu.sync_copy(buf0, acc.at[dst_v.at[j0]], add=True)

            @pl.when(j0 + 2 < CPP)
            def _():
                pltpu.async_copy(g_hbm.at[src_v.at[j0 + 2]], buf0, sem0)

            pltpu.make_async_copy(g_hbm.at[src_v.at[j1]], buf1, sem1).wait()
            pltpu.sync_copy(buf1, acc.at[dst_v.at[j1]], add=True)

            @pl.when(j1 + 2 < CPP)
            def _():
                pltpu.async_copy(g_hbm.at[src_v.at[j1 + 2]], buf1, sem1)

            return carry

        lax.fori_loop(0, CPP // 2, body, 0)

    plsc.subcore_barrier()
    for z in range(ROWS_PT // CHUNKR):
        r0 = s * ROWS_PT + z * CHUNKR
        pltpu.sync_copy(acc.at[pl.ds(r0, CHUNKR)], wb)
        pltpu.sync_copy(wb, out_hbm.at[c, pl.ds(r0, CHUNKR)])


@functools.lru_cache(maxsize=None)
def _seg_kernel_fn():
    return pl.kernel(
        _seg_body,
        out_type=jax.ShapeDtypeStruct((NC, NPADR, D), jnp.float32),
        mesh=_sc_mesh(),
        scratch_types=[
            pltpu.VMEM((CPP, CHUNK), jnp.int32),
            pltpu.VMEM((CPP, CHUNK), jnp.int32),
            pltpu.VMEM((CHUNK, D), jnp.float32),
            pltpu.VMEM((CHUNK, D), jnp.float32),
            pltpu.VMEM_SHARED((NPADR, D), jnp.float32),
            pltpu.SemaphoreType.DMA,
            pltpu.SemaphoreType.DMA,
        ],
    )


def _seg_kernel(src, dst, g, zero_rows):
    return _seg_kernel_fn()(src, dst, g, zero_rows)


def _pre_body(x_ref, w_ref, deg_ref, g_ref, dinv_ref):
    dinv = lax.rsqrt(deg_ref[...])
    h = jnp.dot(x_ref[...], w_ref[...], preferred_element_type=jnp.float32)
    g_ref[...] = h * dinv
    dinv_ref[...] = dinv


def _tc_pre(x, w, deg):
    return pl.pallas_call(
        _pre_body,
        grid=(NB,),
        in_specs=[
            pl.BlockSpec((RB, D), lambda b: (b, 0)),
            pl.BlockSpec((D, D), lambda b: (0, 0)),
            pl.BlockSpec((RB, 1), lambda b: (b, 0)),
        ],
        out_specs=[
            pl.BlockSpec((RB, D), lambda b: (b, 0)),
            pl.BlockSpec((RB, 1), lambda b: (b, 0)),
        ],
        out_shape=[
            jax.ShapeDtypeStruct((N, D), jnp.float32),
            jax.ShapeDtypeStruct((N, 1), jnp.float32),
        ],
    )(x, w, deg)


def _post_body(x_ref, e_ref, g_ref, dinv_ref, bgcn_ref, bnw_ref, bnb_ref,
               w1_ref, b1_ref, w2_ref, b2_ref, out_ref, y_ref, st_ref, cf_ref):
    p = pl.program_id(0)
    b = pl.program_id(1)
    rows = pl.ds(b * RB, RB)

    @pl.when(p == 0)
    def _():
        @pl.when(b == 0)
        def _():
            st_ref[...] = jnp.zeros_like(st_ref)

        e = e_ref[0] + e_ref[1]
        y = x_ref[...] + dinv_ref[...] * (e + g_ref[...]) + bgcn_ref[...]
        y_ref[rows, :] = y
        st_ref[0:1, :] += jnp.sum(y, axis=0, keepdims=True)
        st_ref[1:2, :] += jnp.sum(y * y, axis=0, keepdims=True)

    @pl.when(p == 1)
    def _():
        @pl.when(b == 0)
        def _():
            mean = st_ref[0:1, :] * (1.0 / N)
            var = st_ref[1:2, :] * (1.0 / N) - mean * mean
            sc = bnw_ref[...] * lax.rsqrt(var + EPS)
            cf_ref[0:1, :] = sc
            cf_ref[1:2, :] = bnb_ref[...] - mean * sc

        h1 = y_ref[rows, :] * cf_ref[0:1, :] + cf_ref[1:2, :]
        t = lax.dot_general(h1, w1_ref[...], (((1,), (1,)), ((), ())),
                            preferred_element_type=jnp.float32)
        t = jnp.maximum(t + b1_ref[...], 0.0)
        ff = lax.dot_general(t, w2_ref[...], (((1,), (1,)), ((), ())),
                             preferred_element_type=jnp.float32)
        z = h1 + ff + b2_ref[...]
        y_ref[rows, :] = z
        st_ref[2:3, :] += jnp.sum(z, axis=0, keepdims=True)
        st_ref[3:4, :] += jnp.sum(z * z, axis=0, keepdims=True)

    @pl.when(p == 2)
    def _():
        @pl.when(b == 0)
        def _():
            mean = st_ref[2:3, :] * (1.0 / N)
            var = st_ref[3:4, :] * (1.0 / N) - mean * mean
            sc = bnw_ref[...] * lax.rsqrt(var + EPS)
            cf_ref[2:3, :] = sc
            cf_ref[3:4, :] = bnb_ref[...] - mean * sc

        out_ref[...] = y_ref[rows, :] * cf_ref[2:3, :] + cf_ref[3:4, :]


def _tc_post(x, seg, g, dinv, bgcn, bnw, bnb, w1, b1, w2, b2):
    full = lambda shp: pl.BlockSpec(shp, lambda p, b: tuple(0 for _ in shp))
    return pl.pallas_call(
        _post_body,
        grid=(3, NB),
        in_specs=[
            pl.BlockSpec((RB, D), lambda p, b: (b, 0)),
            pl.BlockSpec((NC, RB, D), lambda p, b: (0, b, 0)),
            pl.BlockSpec((RB, D), lambda p, b: (b, 0)),
            pl.BlockSpec((RB, 1), lambda p, b: (b, 0)),
            full((1, D)),
            full((1, D)),
            full((1, D)),
            full((DFF, D)),
            full((1, DFF)),
            full((D, DFF)),
            full((1, D)),
        ],
        out_specs=pl.BlockSpec((RB, D), lambda p, b: (b, 0)),
        out_shape=jax.ShapeDtypeStruct((N, D), jnp.float32),
        scratch_shapes=[
            pltpu.VMEM((N, D), jnp.float32),
            pltpu.VMEM((8, D), jnp.float32),
            pltpu.VMEM((8, D), jnp.float32),
        ],
    )(x, seg, g, dinv, bgcn, bnw, bnb, w1, b1, w2, b2)


def kernel(x, edge_index, W_gcn, b_gcn, bn_w, bn_b, W1, b1, W2, b2):
    ei = edge_index.astype(jnp.int32)
    src = ei[0].reshape(NW, NCHUNK, CHUNK)
    dst = ei[1].reshape(NW, NCHUNK, CHUNK)
    src2 = ei[0].reshape(NW * NPH, CPP, CHUNK)
    dst2 = ei[1].reshape(NW * NPH, CPP, CHUNK)

    ones_pad = jnp.ones((128,), jnp.float32)
    zeros_deg = jnp.zeros((DEG_PT,), jnp.float32)
    zero_rows = jnp.zeros((CHUNKR, D), jnp.float32)

    deg_p = _deg_kernel(dst, ones_pad, zeros_deg)
    deg = (deg_p[0, :N] + deg_p[1, :N] + 1.0).reshape(N, 1)

    g, dinv = _tc_pre(x, W_gcn, deg)
    seg = _seg_kernel(src2, dst2, g, zero_rows)

    return _tc_post(x, seg, g, dinv,
                    b_gcn.reshape(1, D), bn_w.reshape(1, D), bn_b.reshape(1, D),
                    W1, b1.reshape(1, DFF), W2, b2.reshape(1, D))


# speedup vs baseline: 1.1581x; 1.0008x over previous
import functools

import jax
import jax.numpy as jnp
from jax import lax
from jax.experimental import pallas as pl
from jax.experimental.pallas import tpu as pltpu
from jax.experimental.pallas import tpu_sc as plsc

N = 10000
E = 320000
D = 128
DFF = 512
NC, NS = 2, 16
NW = NC * NS
EPW = E // NW
CHUNK = 100
NCHUNK = EPW // CHUNK
SCHUNK = 50
NBUF = 4
CPP = 40
NPH = EPW // (CPP * SCHUNK)
NPAD = 10240
DEG_PT = NPAD // NS
NPADR = 10240
ROWS_PT = NPADR // NS
CHUNKR = 40
RB = 2000
NB = N // RB
EPS = 1e-5

@functools.lru_cache(maxsize=None)
def _sc_mesh():
    return plsc.VectorSubcoreMesh(
        core_axis_name="c", subcore_axis_name="s", num_cores=NC, num_subcores=NS)


def _deg_body(dst_hbm, ones_hbm, zeros_hbm, deg_hbm, idx_v, ones_v, zb_v, acc,
              dsem):
    c = lax.axis_index("c")
    s = lax.axis_index("s")
    wid = c * NS + s

    pltpu.sync_copy(zeros_hbm, zb_v)
    pltpu.sync_copy(zb_v, acc.at[pl.ds(s * DEG_PT, DEG_PT)])
    pltpu.sync_copy(dst_hbm.at[wid], idx_v)
    pltpu.sync_copy(ones_hbm.at[pl.ds(0, CHUNK)], ones_v)
    plsc.subcore_barrier()

    def body(j, carry):
        pltpu.async_copy(ones_v, acc.at[idx_v.at[j]], dsem, add=True)
        return carry

    lax.fori_loop(0, NCHUNK, body, 0)

    def drain(j, carry):
        pltpu.make_async_copy(ones_v, acc.at[idx_v.at[j]], dsem).wait()
        return carry

    lax.fori_loop(0, NCHUNK, drain, 0)
    plsc.subcore_barrier()
    pltpu.sync_copy(acc.at[pl.ds(s * DEG_PT, DEG_PT)], zb_v)
    pltpu.sync_copy(zb_v, deg_hbm.at[c, pl.ds(s * DEG_PT, DEG_PT)])


@functools.lru_cache(maxsize=None)
def _deg_kernel_fn():
    return pl.kernel(
        _deg_body,
        out_type=jax.ShapeDtypeStruct((NC, NPAD), jnp.float32),
        mesh=_sc_mesh(),
        scratch_types=[
            pltpu.VMEM((NCHUNK, CHUNK), jnp.int32),
            pltpu.VMEM((CHUNK,), jnp.float32),
            pltpu.VMEM((DEG_PT,), jnp.float32),
            pltpu.VMEM_SHARED((NPAD,), jnp.float32),
            pltpu.SemaphoreType.DMA,
        ],
    )


def _deg_kernel(dst, ones_pad, zeros_deg):
    return _deg_kernel_fn()(dst, ones_pad, zeros_deg)


def _seg_body(src_hbm, dst_hbm, g_hbm, zrow_hbm, out_hbm,
              src_v, dst_v, buf0, buf1, buf2, buf3, acc, sem0, sem1, sem2,
              sem3):
    c = lax.axis_index("c")
    s = lax.axis_index("s")
    wid = c * NS + s
    bufs = (buf0, buf1, buf2, buf3)
    sems = (sem0, sem1, sem2, sem3)

    wb = buf0.at[pl.ds(0, CHUNKR)]
    pltpu.sync_copy(zrow_hbm, wb)
    for z in range(ROWS_PT // CHUNKR):
        pltpu.sync_copy(wb, acc.at[pl.ds(s * ROWS_PT + z * CHUNKR, CHUNKR)])
    plsc.subcore_barrier()

    for ph in range(NPH):
        blk = wid * NPH + ph
        pltpu.sync_copy(src_hbm.at[blk], src_v)
        pltpu.sync_copy(dst_hbm.at[blk], dst_v)
        for u in range(NBUF - 1):
            pltpu.async_copy(g_hbm.at[src_v.at[u]], bufs[u], sems[u])

        def body(k, carry):
            for u in range(NBUF):
                j = NBUF * k + u
                pltpu.make_async_copy(
                    g_hbm.at[src_v.at[j]], bufs[u], sems[u]).wait()
                pltpu.sync_copy(bufs[u], acc.at[dst_v.at[j]], add=True)
                un = (u + NBUF - 1) % NBUF

                @pl.when(j + NBUF - 1 < CPP)
                def _():
                    pltpu.async_copy(
                        g_hbm.at[src_v.at[j + NBUF - 1]], bufs[un], sems[un])

            return carry

        lax.fori_loop(0, CPP // NBUF, body, 0)

    plsc.subcore_barrier()
    for z in range(ROWS_PT // CHUNKR):
        r0 = s * ROWS_PT + z * CHUNKR
        pltpu.sync_copy(acc.at[pl.ds(r0, CHUNKR)], wb)
        pltpu.sync_copy(wb, out_hbm.at[c, pl.ds(r0, CHUNKR)])


@functools.lru_cache(maxsize=None)
def _seg_kernel_fn():
    return pl.kernel(
        _seg_body,
        out_type=jax.ShapeDtypeStruct((NC, NPADR, D), jnp.float32),
        mesh=_sc_mesh(),
        scratch_types=[
            pltpu.VMEM((CPP, SCHUNK), jnp.int32),
            pltpu.VMEM((CPP, SCHUNK), jnp.int32),
            pltpu.VMEM((SCHUNK, D), jnp.float32),
            pltpu.VMEM((SCHUNK, D), jnp.float32),
            pltpu.VMEM((SCHUNK, D), jnp.float32),
            pltpu.VMEM((SCHUNK, D), jnp.float32),
            pltpu.VMEM_SHARED((NPADR, D), jnp.float32),
            pltpu.SemaphoreType.DMA,
            pltpu.SemaphoreType.DMA,
            pltpu.SemaphoreType.DMA,
            pltpu.SemaphoreType.DMA,
        ],
    )


def _seg_kernel(src, dst, g, zero_rows):
    return _seg_kernel_fn()(src, dst, g, zero_rows)


def _pre_body(x_ref, w_ref, deg_ref, g_ref, dinv_ref):
    dinv = lax.rsqrt(deg_ref[...])
    h = jnp.dot(x_ref[...], w_ref[...], preferred_element_type=jnp.float32)
    g_ref[...] = h * dinv
    dinv_ref[...] = dinv


def _tc_pre(x, w, deg):
    return pl.pallas_call(
        _pre_body,
        grid=(NB,),
        in_specs=[
            pl.BlockSpec((RB, D), lambda b: (b, 0)),
            pl.BlockSpec((D, D), lambda b: (0, 0)),
            pl.BlockSpec((RB, 1), lambda b: (b, 0)),
        ],
        out_specs=[
            pl.BlockSpec((RB, D), lambda b: (b, 0)),
            pl.BlockSpec((RB, 1), lambda b: (b, 0)),
        ],
        out_shape=[
            jax.ShapeDtypeStruct((N, D), jnp.float32),
            jax.ShapeDtypeStruct((N, 1), jnp.float32),
        ],
    )(x, w, deg)


def _post_body(x_ref, e_ref, g_ref, dinv_ref, bgcn_ref, bnw_ref, bnb_ref,
               w1_ref, b1_ref, w2_ref, b2_ref, out_ref, y_ref, st_ref, cf_ref):
    p = pl.program_id(0)
    b = pl.program_id(1)
    rows = pl.ds(b * RB, RB)

    @pl.when(p == 0)
    def _():
        @pl.when(b == 0)
        def _():
            st_ref[...] = jnp.zeros_like(st_ref)

        e = e_ref[0] + e_ref[1]
        y = x_ref[...] + dinv_ref[...] * (e + g_ref[...]) + bgcn_ref[...]
        y_ref[rows, :] = y
        st_ref[0:1, :] += jnp.sum(y, axis=0, keepdims=True)
        st_ref[1:2, :] += jnp.sum(y * y, axis=0, keepdims=True)

    @pl.when(p == 1)
    def _():
        @pl.when(b == 0)
        def _():
            mean = st_ref[0:1, :] * (1.0 / N)
            var = st_ref[1:2, :] * (1.0 / N) - mean * mean
            sc = bnw_ref[...] * lax.rsqrt(var + EPS)
            cf_ref[0:1, :] = sc
            cf_ref[1:2, :] = bnb_ref[...] - mean * sc

        h1 = y_ref[rows, :] * cf_ref[0:1, :] + cf_ref[1:2, :]
        t = lax.dot_general(h1, w1_ref[...], (((1,), (1,)), ((), ())),
                            preferred_element_type=jnp.float32)
        t = jnp.maximum(t + b1_ref[...], 0.0)
        ff = lax.dot_general(t, w2_ref[...], (((1,), (1,)), ((), ())),
                             preferred_element_type=jnp.float32)
        z = h1 + ff + b2_ref[...]
        y_ref[rows, :] = z
        st_ref[2:3, :] += jnp.sum(z, axis=0, keepdims=True)
        st_ref[3:4, :] += jnp.sum(z * z, axis=0, keepdims=True)

    @pl.when(p == 2)
    def _():
        @pl.when(b == 0)
        def _():
            mean = st_ref[2:3, :] * (1.0 / N)
            var = st_ref[3:4, :] * (1.0 / N) - mean * mean
            sc = bnw_ref[...] * lax.rsqrt(var + EPS)
            cf_ref[2:3, :] = sc
            cf_ref[3:4, :] = bnb_ref[...] - mean * sc

        out_ref[...] = y_ref[rows, :] * cf_ref[2:3, :] + cf_ref[3:4, :]


def _tc_post(x, seg, g, dinv, bgcn, bnw, bnb, w1, b1, w2, b2):
    full = lambda shp: pl.BlockSpec(shp, lambda p, b: tuple(0 for _ in shp))
    return pl.pallas_call(
        _post_body,
        grid=(3, NB),
        in_specs=[
            pl.BlockSpec((RB, D), lambda p, b: (b, 0)),
            pl.BlockSpec((NC, RB, D), lambda p, b: (0, b, 0)),
            pl.BlockSpec((RB, D), lambda p, b: (b, 0)),
            pl.BlockSpec((RB, 1), lambda p, b: (b, 0)),
            full((1, D)),
            full((1, D)),
            full((1, D)),
            full((DFF, D)),
            full((1, DFF)),
            full((D, DFF)),
            full((1, D)),
        ],
        out_specs=pl.BlockSpec((RB, D), lambda p, b: (b, 0)),
        out_shape=jax.ShapeDtypeStruct((N, D), jnp.float32),
        scratch_shapes=[
            pltpu.VMEM((N, D), jnp.float32),
            pltpu.VMEM((8, D), jnp.float32),
            pltpu.VMEM((8, D), jnp.float32),
        ],
    )(x, seg, g, dinv, bgcn, bnw, bnb, w1, b1, w2, b2)


def kernel(x, edge_index, W_gcn, b_gcn, bn_w, bn_b, W1, b1, W2, b2):
    ei = edge_index.astype(jnp.int32)
    src = ei[0].reshape(NW, NCHUNK, CHUNK)
    dst = ei[1].reshape(NW, NCHUNK, CHUNK)
    src2 = ei[0].reshape(NW * NPH, CPP, SCHUNK)
    dst2 = ei[1].reshape(NW * NPH, CPP, SCHUNK)

    ones_pad = jnp.ones((128,), jnp.float32)
    zeros_deg = jnp.zeros((DEG_PT,), jnp.float32)
    zero_rows = jnp.zeros((CHUNKR, D), jnp.float32)

    deg_p = _deg_kernel(dst, ones_pad, zeros_deg)
    deg = (deg_p[0, :N] + deg_p[1, :N] + 1.0).reshape(N, 1)

    g, dinv = _tc_pre(x, W_gcn, deg)
    seg = _seg_kernel(src2, dst2, g, zero_rows)

    return _tc_post(x, seg, g, dinv,
                    b_gcn.reshape(1, D), bn_w.reshape(1, D), bn_b.reshape(1, D),
                    W1, b1.reshape(1, DFF), W2, b2.reshape(1, D))


# speedup vs baseline: 1.2200x; 1.0534x over previous
import functools

import jax
import jax.numpy as jnp
from jax import lax
from jax.experimental import pallas as pl
from jax.experimental.pallas import tpu as pltpu
from jax.experimental.pallas import tpu_sc as plsc

N = 10000
E = 320000
D = 128
DFF = 512
NC, NS = 2, 16
NW = NC * NS
EPW = E // NW
CHUNK = 100
NCHUNK = EPW // CHUNK
SCHUNK = 50
NBUF = 4
CPP = 40
NPH = EPW // (CPP * SCHUNK)
NPAD = 10240
DEG_PT = NPAD // NS
NPADR = 10240
ROWS_PT = NPADR // NS
CHUNKR = 40
RB = 2000
NB = N // RB
EPS = 1e-5

@functools.lru_cache(maxsize=None)
def _sc_mesh():
    return plsc.VectorSubcoreMesh(
        core_axis_name="c", subcore_axis_name="s", num_cores=NC, num_subcores=NS)


def _deg_body(dst_hbm, ones_hbm, zeros_hbm, deg_hbm, idx_v, ones_v, zb_v, acc,
              dsem):
    c = lax.axis_index("c")
    s = lax.axis_index("s")
    wid = c * NS + s

    pltpu.sync_copy(zeros_hbm, zb_v)
    pltpu.sync_copy(zb_v, acc.at[pl.ds(s * DEG_PT, DEG_PT)])
    pltpu.sync_copy(dst_hbm.at[wid], idx_v)
    pltpu.sync_copy(ones_hbm.at[pl.ds(0, CHUNK)], ones_v)
    plsc.subcore_barrier()

    def body(j, carry):
        pltpu.async_copy(ones_v, acc.at[idx_v.at[j]], dsem, add=True)
        return carry

    lax.fori_loop(0, NCHUNK, body, 0)

    def drain(j, carry):
        pltpu.make_async_copy(ones_v, acc.at[idx_v.at[j]], dsem).wait()
        return carry

    lax.fori_loop(0, NCHUNK, drain, 0)
    plsc.subcore_barrier()
    pltpu.sync_copy(acc.at[pl.ds(s * DEG_PT, DEG_PT)], zb_v)
    pltpu.sync_copy(zb_v, deg_hbm.at[c, pl.ds(s * DEG_PT, DEG_PT)])


@functools.lru_cache(maxsize=None)
def _deg_kernel_fn():
    return pl.kernel(
        _deg_body,
        out_type=jax.ShapeDtypeStruct((NC, NPAD), jnp.float32),
        mesh=_sc_mesh(),
        scratch_types=[
            pltpu.VMEM((NCHUNK, CHUNK), jnp.int32),
            pltpu.VMEM((CHUNK,), jnp.float32),
            pltpu.VMEM((DEG_PT,), jnp.float32),
            pltpu.VMEM_SHARED((NPAD,), jnp.float32),
            pltpu.SemaphoreType.DMA,
        ],
    )


def _deg_kernel(dst, ones_pad, zeros_deg):
    return _deg_kernel_fn()(dst, ones_pad, zeros_deg)


def _seg_body(src_hbm, dst_hbm, g_hbm, zrow_hbm, out_hbm,
              src_v, dst_v, buf0, buf1, buf2, buf3, acc, sem0, sem1, sem2,
              sem3):
    c = lax.axis_index("c")
    s = lax.axis_index("s")
    wid = c * NS + s
    bufs = (buf0, buf1, buf2, buf3)
    sems = (sem0, sem1, sem2, sem3)

    wb = buf0.at[pl.ds(0, CHUNKR)]
    pltpu.sync_copy(zrow_hbm, wb)
    for z in range(ROWS_PT // CHUNKR):
        pltpu.sync_copy(wb, acc.at[pl.ds(s * ROWS_PT + z * CHUNKR, CHUNKR)])
    plsc.subcore_barrier()

    for ph in range(NPH):
        blk = wid * NPH + ph
        pltpu.sync_copy(src_hbm.at[blk], src_v)
        pltpu.sync_copy(dst_hbm.at[blk], dst_v)
        for u in range(NBUF - 1):
            pltpu.async_copy(g_hbm.at[src_v.at[u]], bufs[u], sems[u])

        def body(k, carry):
            for u in range(NBUF):
                j = NBUF * k + u
                pltpu.make_async_copy(
                    g_hbm.at[src_v.at[j]], bufs[u], sems[u]).wait()
                pltpu.sync_copy(bufs[u], acc.at[dst_v.at[j]], add=True)
                un = (u + NBUF - 1) % NBUF

                @pl.when(j + NBUF - 1 < CPP)
                def _():
                    pltpu.async_copy(
                        g_hbm.at[src_v.at[j + NBUF - 1]], bufs[un], sems[un])

            return carry

        lax.fori_loop(0, CPP // NBUF, body, 0)

    plsc.subcore_barrier()
    for z in range(ROWS_PT // CHUNKR):
        r0 = s * ROWS_PT + z * CHUNKR
        pltpu.sync_copy(acc.at[pl.ds(r0, CHUNKR)], wb)
        pltpu.sync_copy(wb, out_hbm.at[c, pl.ds(r0, CHUNKR)])


@functools.lru_cache(maxsize=None)
def _seg_kernel_fn():
    return pl.kernel(
        _seg_body,
        out_type=jax.ShapeDtypeStruct((NC, NPADR, D), jnp.float32),
        mesh=_sc_mesh(),
        scratch_types=[
            pltpu.VMEM((CPP, SCHUNK), jnp.int32),
            pltpu.VMEM((CPP, SCHUNK), jnp.int32),
            pltpu.VMEM((SCHUNK, D), jnp.float32),
            pltpu.VMEM((SCHUNK, D), jnp.float32),
            pltpu.VMEM((SCHUNK, D), jnp.float32),
            pltpu.VMEM((SCHUNK, D), jnp.float32),
            pltpu.VMEM_SHARED((NPADR, D), jnp.float32),
            pltpu.SemaphoreType.DMA,
            pltpu.SemaphoreType.DMA,
            pltpu.SemaphoreType.DMA,
            pltpu.SemaphoreType.DMA,
        ],
    )


def _seg_kernel(src, dst, g, zero_rows):
    return _seg_kernel_fn()(src, dst, g, zero_rows)


def _pre_body(x_ref, w_ref, deg_ref, g_ref, dinv_ref):
    dinv = lax.rsqrt(deg_ref[...])
    h = jnp.dot(x_ref[...], w_ref[...], preferred_element_type=jnp.float32)
    g_ref[...] = h * dinv
    dinv_ref[...] = dinv


def _tc_pre(x, w, deg):
    return pl.pallas_call(
        _pre_body,
        grid=(NB,),
        in_specs=[
            pl.BlockSpec((RB, D), lambda b: (b, 0)),
            pl.BlockSpec((D, D), lambda b: (0, 0)),
            pl.BlockSpec((RB, 1), lambda b: (b, 0)),
        ],
        out_specs=[
            pl.BlockSpec((RB, D), lambda b: (b, 0)),
            pl.BlockSpec((RB, 1), lambda b: (b, 0)),
        ],
        out_shape=[
            jax.ShapeDtypeStruct((N, D), jnp.float32),
            jax.ShapeDtypeStruct((N, 1), jnp.float32),
        ],
    )(x, w, deg)


def _post_body(x_ref, e_ref, g_ref, dinv_ref, bgcn_ref, bnw_ref, bnb_ref,
               w1_ref, b1_ref, w2_ref, b2_ref, out_ref, y_ref, st_ref, cf_ref):
    p = pl.program_id(0)
    b = pl.program_id(1)
    rows = pl.ds(b * RB, RB)

    @pl.when(p == 0)
    def _():
        @pl.when(b == 0)
        def _():
            st_ref[...] = jnp.zeros_like(st_ref)

        e = e_ref[0] + e_ref[1]
        y = x_ref[...] + dinv_ref[...] * (e + g_ref[...]) + bgcn_ref[...]
        y_ref[rows, :] = y
        st_ref[0:1, :] += jnp.sum(y, axis=0, keepdims=True)
        st_ref[1:2, :] += jnp.sum(y * y, axis=0, keepdims=True)

    @pl.when(p == 1)
    def _():
        @pl.when(b == 0)
        def _():
            mean = st_ref[0:1, :] * (1.0 / N)
            var = st_ref[1:2, :] * (1.0 / N) - mean * mean
            sc = bnw_ref[...] * lax.rsqrt(var + EPS)
            cf_ref[0:1, :] = sc
            cf_ref[1:2, :] = bnb_ref[...] - mean * sc

        h1 = y_ref[rows, :] * cf_ref[0:1, :] + cf_ref[1:2, :]
        t = lax.dot_general(h1, w1_ref[...], (((1,), (1,)), ((), ())),
                            preferred_element_type=jnp.float32)
        t = jnp.maximum(t + b1_ref[...], 0.0)
        ff = lax.dot_general(t, w2_ref[...], (((1,), (1,)), ((), ())),
                             preferred_element_type=jnp.float32)
        z = h1 + ff + b2_ref[...]
        y_ref[rows, :] = z
        st_ref[2:3, :] += jnp.sum(z, axis=0, keepdims=True)
        st_ref[3:4, :] += jnp.sum(z * z, axis=0, keepdims=True)

    @pl.when(p == 2)
    def _():
        @pl.when(b == 0)
        def _():
            mean = st_ref[2:3, :] * (1.0 / N)
            var = st_ref[3:4, :] * (1.0 / N) - mean * mean
            sc = bnw_ref[...] * lax.rsqrt(var + EPS)
            cf_ref[2:3, :] = sc
            cf_ref[3:4, :] = bnb_ref[...] - mean * sc

        out_ref[...] = y_ref[rows, :] * cf_ref[2:3, :] + cf_ref[3:4, :]


def _tc_post(x, seg, g, dinv, bgcn, bnw, bnb, w1, b1, w2, b2):
    full = lambda shp: pl.BlockSpec(shp, lambda p, b: tuple(0 for _ in shp))
    return pl.pallas_call(
        _post_body,
        grid=(3, NB),
        in_specs=[
            pl.BlockSpec((RB, D), lambda p, b: (jnp.where(p == 0, b, 0), 0)),
            pl.BlockSpec((NC, RB, D),
                         lambda p, b: (0, jnp.where(p == 0, b, 0), 0)),
            pl.BlockSpec((RB, D), lambda p, b: (jnp.where(p == 0, b, 0), 0)),
            pl.BlockSpec((RB, 1), lambda p, b: (jnp.where(p == 0, b, 0), 0)),
            full((1, D)),
            full((1, D)),
            full((1, D)),
            full((DFF, D)),
            full((1, DFF)),
            full((D, DFF)),
            full((1, D)),
        ],
        out_specs=pl.BlockSpec((RB, D),
                               lambda p, b: (jnp.where(p == 2, b, 0), 0)),
        out_shape=jax.ShapeDtypeStruct((N, D), jnp.float32),
        scratch_shapes=[
            pltpu.VMEM((N, D), jnp.float32),
            pltpu.VMEM((8, D), jnp.float32),
            pltpu.VMEM((8, D), jnp.float32),
        ],
    )(x, seg, g, dinv, bgcn, bnw, bnb, w1, b1, w2, b2)


def kernel(x, edge_index, W_gcn, b_gcn, bn_w, bn_b, W1, b1, W2, b2):
    ei = edge_index.astype(jnp.int32)
    src = ei[0].reshape(NW, NCHUNK, CHUNK)
    dst = ei[1].reshape(NW, NCHUNK, CHUNK)
    src2 = ei[0].reshape(NW * NPH, CPP, SCHUNK)
    dst2 = ei[1].reshape(NW * NPH, CPP, SCHUNK)

    ones_pad = jnp.ones((128,), jnp.float32)
    zeros_deg = jnp.zeros((DEG_PT,), jnp.float32)
    zero_rows = jnp.zeros((CHUNKR, D), jnp.float32)

    deg_p = _deg_kernel(dst, ones_pad, zeros_deg)
    deg = (deg_p[0, :N] + deg_p[1, :N] + 1.0).reshape(N, 1)

    g, dinv = _tc_pre(x, W_gcn, deg)
    seg = _seg_kernel(src2, dst2, g, zero_rows)

    return _tc_post(x, seg, g, dinv,
                    b_gcn.reshape(1, D), bn_w.reshape(1, D), bn_b.reshape(1, D),
                    W1, b1.reshape(1, DFF), W2, b2.reshape(1, D))
